# Initial kernel scaffold; baseline (speedup 1.0000x reference)
import jax
import jax.numpy as jnp
from jax.experimental import pallas as pl


def kernel(nf, ei, ew, W1, as1, ad1, We1, ae1, b1, W2, as2, ad2, We2, ae2, b2, g1, bn1, g2, bn2, g3, bn3, Wf1, bf1, Wf2, bf2):
    raise NotImplementedError("write your pallas kernel here")


# speedup vs baseline: 34.3741x; 34.3741x over previous
import functools
import jax
import jax.numpy as jnp
from jax import lax
from jax.experimental import pallas as pl
from jax.experimental.pallas import tpu as pltpu
from jax.experimental.pallas import tpu_sc as plsc

N = 10000
E = 320000
D = 128
H = 8
C = 16
DE = 16
DFF = 512

NPAD = 10240
ROW = 160
K = 80
TILE_E = E // 32
NCHUNK = TILE_E // K
NB = 1000
EB = 4000


def _mm_body(x_ref, w_ref, o_ref):
    o_ref[...] = jnp.dot(x_ref[...], w_ref[...],
                         preferred_element_type=jnp.float32)


def _proj(nf, wcat):
    return pl.pallas_call(
        _mm_body,
        grid=(N // NB,),
        in_specs=[pl.BlockSpec((NB, D), lambda i: (i, 0)),
                  pl.BlockSpec((D, ROW), lambda i: (0, 0))],
        out_specs=pl.BlockSpec((NB, ROW), lambda i: (i, 0)),
        out_shape=jax.ShapeDtypeStruct((N, ROW), jnp.float32),
    )(nf, wcat)


def _ale_body(ew_ref, v1_ref, v2_ref, o1_ref, o2_ref):
    ewb = ew_ref[...]
    o1_ref[...] = jnp.dot(ewb, v1_ref[...], preferred_element_type=jnp.float32)
    o2_ref[...] = jnp.dot(ewb, v2_ref[...], preferred_element_type=jnp.float32)


def _ale_both(ew, ve1, ve2):
    return pl.pallas_call(
        _ale_body,
        grid=(E // EB,),
        in_specs=[pl.BlockSpec((EB, DE), lambda i: (i, 0)),
                  pl.BlockSpec((DE, 16), lambda i: (0, 0)),
                  pl.BlockSpec((DE, 16), lambda i: (0, 0))],
        out_specs=[pl.BlockSpec((EB, 16), lambda i: (i, 0)),
                   pl.BlockSpec((EB, 16), lambda i: (i, 0))],
        out_shape=[jax.ShapeDtypeStruct((E, 16), jnp.float32),
                   jax.ShapeDtypeStruct((E, 16), jnp.float32)],
    )(ew, ve1, ve2)


def _post_body(p0_ref, p1_ref, g_ref, nf_ref, b_ref, gg_ref, bn_ref, p8_ref,
               o_ref):
    p0 = p0_ref[0]
    p1 = p1_ref[0]
    g = g_ref[...]
    h = g[:, :D]
    als = g[:, D:D + H]
    ald = g[:, D + 16:D + 16 + H]
    acc = p0[:, :D] + p1[:, :D]
    den_p = p0[:, D:D + H] + p1[:, D:D + H]
    acc_la = p0[:, D + H:D + 16] + p1[:, D + H:D + 16]
    deg = p0[:, D + 16:D + 17] + p1[:, D + 16:D + 17]
    ale_loop = acc_la / jnp.maximum(deg, 1.0)
    al = als + ald + ale_loop
    al = jnp.where(al > 0, al, 0.2 * al)
    exl = jnp.exp(al)
    rden = 1.0 / (den_p + exl + 1e-16)
    p8 = p8_ref[...]
    exl128 = jnp.dot(exl, p8, preferred_element_type=jnp.float32)
    rden128 = jnp.dot(rden, p8, preferred_element_type=jnp.float32)
    a1 = (acc + exl128 * h) * rden128 + b_ref[...]
    m = jnp.mean(a1, axis=1, keepdims=True)
    xc = a1 - m
    s = jnp.sqrt(jnp.sum(xc * xc, axis=1, keepdims=True) / (D - 1))
    y = gg_ref[...] * xc / (s + 1e-6) + bn_ref[...]
    y = jnp.where(y > 0, y, 0.01 * y)
    o_ref[...] = nf_ref[...] + y


def _post(part, g, nf, b, gg, bn, p8):
    return pl.pallas_call(
        _post_body,
        grid=(N // NB,),
        in_specs=[pl.BlockSpec((1, NB, ROW), lambda i: (0, i, 0)),
                  pl.BlockSpec((1, NB, ROW), lambda i: (1, i, 0)),
                  pl.BlockSpec((NB, ROW), lambda i: (i, 0)),
                  pl.BlockSpec((NB, D), lambda i: (i, 0)),
                  pl.BlockSpec((1, D), lambda i: (0, 0)),
                  pl.BlockSpec((1, D), lambda i: (0, 0)),
                  pl.BlockSpec((1, D), lambda i: (0, 0)),
                  pl.BlockSpec((H, D), lambda i: (0, 0))],
        out_specs=pl.BlockSpec((NB, D), lambda i: (i, 0)),
        out_shape=jax.ShapeDtypeStruct((N, D), jnp.float32),
    )(part, part, g, nf, b, gg, bn, p8)


def _ffn_body(nf_ref, w1_ref, b1_ref, w2_ref, b2_ref, gg_ref, bn_ref, o_ref):
    nf = nf_ref[...]
    t = jnp.dot(nf, w1_ref[...], preferred_element_type=jnp.float32)
    t = jnp.maximum(t + b1_ref[...], 0.0)
    ff = jnp.dot(t, w2_ref[...], preferred_element_type=jnp.float32)
    ff = ff + b2_ref[...]
    m = jnp.mean(ff, axis=1, keepdims=True)
    xc = ff - m
    s = jnp.sqrt(jnp.sum(xc * xc, axis=1, keepdims=True) / (D - 1))
    y = gg_ref[...] * xc / (s + 1e-6) + bn_ref[...]
    y = jnp.where(y > 0, y, 0.01 * y)
    o_ref[...] = nf + y


def _ffn(nf, w1, b1, w2, b2, gg, bn):
    return pl.pallas_call(
        _ffn_body,
        grid=(N // NB,),
        in_specs=[pl.BlockSpec((NB, D), lambda i: (i, 0)),
                  pl.BlockSpec((D, DFF), lambda i: (0, 0)),
                  pl.BlockSpec((1, DFF), lambda i: (0, 0)),
                  pl.BlockSpec((DFF, D), lambda i: (0, 0)),
                  pl.BlockSpec((1, D), lambda i: (0, 0)),
                  pl.BlockSpec((1, D), lambda i: (0, 0)),
                  pl.BlockSpec((1, D), lambda i: (0, 0))],
        out_specs=pl.BlockSpec((NB, D), lambda i: (i, 0)),
        out_shape=jax.ShapeDtypeStruct((N, D), jnp.float32),
    )(nf, w1, b1, w2, b2, gg, bn)


def _sc_body(src_h, dst_h, ale_h, g_h, ad_h, part_h,
             idxs_v, idxd_v, grows_v, adst_v, alev_v, s_v, acc_sh):
    cid = lax.axis_index("c")
    sid = lax.axis_index("s")
    lanes = lax.iota(jnp.int32, 16)
    zv = jnp.zeros((16,), jnp.float32)
    onesv = jnp.where(lanes < 1, 1.0, 0.0).astype(jnp.float32)
    lo8 = lanes < 8

    def zrow(i, _):
        r = i // 10
        col = (i % 10) * 16
        s_v[r, pl.ds(col, 16)] = zv
        return 0
    lax.fori_loop(0, K * 10, zrow, 0)

    def zcopy(j, _):
        pltpu.sync_copy(s_v, acc_sh.at[pl.ds(sid * 640 + j * K, K)])
        return 0
    lax.fori_loop(0, 640 // K, zcopy, 0)
    plsc.subcore_barrier()

    estart = cid * (E // 2) + sid * TILE_E

    def chunk(c, _):
        base = estart + c * K
        pltpu.sync_copy(src_h.at[pl.ds(base, K)], idxs_v)
        pltpu.sync_copy(dst_h.at[pl.ds(base, K)], idxd_v)
        pltpu.sync_copy(ale_h.at[pl.ds(base, K)], alev_v)
        pltpu.sync_copy(g_h.at[idxs_v], grows_v)
        pltpu.sync_copy(ad_h.at[idxd_v], adst_v)

        def edge(e, _):
            av = grows_v[e, pl.ds(D, 16)] + adst_v[e, :] + alev_v[e, :]
            al = jnp.where(av > 0, av, 0.2 * av)
            ex = jnp.exp(al)
            mix = jnp.where(lo8, ex, alev_v[e, :])
            s_v[e, pl.ds(D, 16)] = mix
            s_v[e, pl.ds(D + 16, 16)] = onesv
            for hh in range(H):
                s_v[e, pl.ds(hh * 16, 16)] = (
                    grows_v[e, pl.ds(hh * 16, 16)] * ex[hh])
            return 0
        lax.fori_loop(0, K, edge, 0)
        pltpu.sync_copy(s_v, acc_sh.at[idxd_v], add=True)
        return 0
    lax.fori_loop(0, NCHUNK, chunk, 0)
    plsc.subcore_barrier()
    pltpu.sync_copy(acc_sh.at[pl.ds(sid * 640, 640)],
                    part_h.at[cid, pl.ds(sid * 640, 640)])


def _sc_edge_pass(src, dst, ale, g, ad):
    mesh = plsc.VectorSubcoreMesh(core_axis_name="c", subcore_axis_name="s")
    f = pl.kernel(
        _sc_body,
        mesh=mesh,
        compiler_params=pltpu.CompilerParams(use_tc_tiling_on_sc=False),
        out_type=jax.ShapeDtypeStruct((2, NPAD, ROW), jnp.float32),
        scratch_types=[
            pltpu.VMEM((K,), jnp.int32),
            pltpu.VMEM((K,), jnp.int32),
            pltpu.VMEM((K, ROW), jnp.float32),
            pltpu.VMEM((K, 16), jnp.float32),
            pltpu.VMEM((K, 16), jnp.float32),
            pltpu.VMEM((K, ROW), jnp.float32),
            pltpu.VMEM_SHARED((NPAD, ROW), jnp.float32),
        ],
    )
    return f(src, dst, ale, g, ad)


def _prep_w(W, a_src, a_dst):
    w3 = W.reshape(D, H, C)
    ws = jnp.einsum('dhc,hc->dh', w3, a_src)
    wd = jnp.einsum('dhc,hc->dh', w3, a_dst)
    return jnp.concatenate([W, ws, ws, wd, wd], axis=1)


def _prep_ve(We, a_e):
    ve = jnp.einsum('dhc,hc->dh', We.reshape(DE, H, C), a_e)
    return jnp.concatenate([ve, ve], axis=1)


def kernel(nf, ei, ew, W1, as1, ad1, We1, ae1, b1, W2, as2, ad2, We2, ae2, b2,
           g1, bn1, g2, bn2, g3, bn3, Wf1, bf1, Wf2, bf2):
    src = ei[0]
    dst = ei[1]
    wcat1 = _prep_w(W1, as1, ad1)
    wcat2 = _prep_w(W2, as2, ad2)
    ale1, ale2 = _ale_both(ew, _prep_ve(We1, ae1), _prep_ve(We2, ae2))
    p8 = jnp.repeat(jnp.eye(H, dtype=jnp.float32), C, axis=1)

    def gat(nf_in, wcat, ale, bias, gg, bn):
        g = _proj(nf_in, wcat)
        ad = g[:, D + 16:]
        part = _sc_edge_pass(src, dst, ale, g, ad)
        return _post(part, g, nf_in,
                     bias.reshape(1, D), gg.reshape(1, D), bn.reshape(1, D),
                     p8)

    nf = gat(nf, wcat1, ale1, b1, g1, bn1)
    nf = gat(nf, wcat2, ale2, b2, g2, bn2)
    nf = _ffn(nf, Wf1, bf1.reshape(1, DFF), Wf2, bf2.reshape(1, D),
              g3.reshape(1, D), bn3.reshape(1, D))
    return nf
